# baseline (speedup 1.0000x reference)
import functools

import jax
import jax.numpy as jnp
from jax.experimental import pallas as pl
from jax.experimental.pallas import tpu as pltpu
from jax.experimental.pallas import tpu_sc as plsc

_K = 16
_KP1 = 17
_N = 2048
_D = 64
_R = 512
_NK = _N * _K


def _topk_kernel(xt_ref, xq_ref, sqc_ref, sqr_ref, idx_ref):
    xt = xt_ref[0]
    xq = xq_ref[0]
    mm = jax.lax.dot_general(
        xt, xq, (((1,), (0,)), ((), ())),
        preferred_element_type=jnp.float32)
    d2 = (sqc_ref[0] + sqr_ref[0]) - 2.0 * mm
    dist = jnp.sqrt(jnp.maximum(d2, 0.0))
    iota = jax.lax.broadcasted_iota(jnp.int32, (_N, _R), 0)
    iota8 = jax.lax.broadcasted_iota(jnp.int32, (8, _R), 0)
    dT = dist
    nch = _N // 8
    nchain = 4
    cpc = nch // nchain
    for t in range(_KP1):
        vs, cs = [], []
        for ch in range(nchain):
            b0 = ch * cpc
            vacc = dT[b0 * 8:(b0 + 1) * 8, :]
            cacc = jnp.full((8, _R), b0, jnp.int32)
            for c in range(b0 + 1, b0 + cpc):
                v = dT[c * 8:(c + 1) * 8, :]
                take = v < vacc
                vacc = jnp.where(take, v, vacc)
                cacc = jnp.where(take, c, cacc)
            vs.append(vacc)
            cs.append(cacc)
        while len(vs) > 1:
            take = vs[1] < vs[0]
            vs = [jnp.where(take, vs[1], vs[0])] + vs[2:]
            cs = [jnp.where(take, cs[1], cs[0])] + cs[2:]
        fidx = cs[0] * 8 + iota8
        vcur, icur = vs[0], fidx
        for sh in (4, 2, 1):
            vb = pltpu.roll(vcur, 8 - sh, axis=0)
            ib = pltpu.bitcast(
                pltpu.roll(pltpu.bitcast(icur, jnp.float32), 8 - sh, axis=0),
                jnp.int32)
            take = (vb < vcur) | ((vb == vcur) & (ib < icur))
            vcur = jnp.where(take, vb, vcur)
            icur = jnp.where(take, ib, icur)
        j = icur[0:1, :]
        hit = iota == j
        if t >= 1:
            idx_ref[0, t - 1:t, :] = j
        if t < _KP1 - 1:
            dT = jnp.where(hit, jnp.float32(jnp.inf), dT)


def _sc_gather_kernel(x_hbm, idxf_hbm, out_hbm, table_v, idx_v, cen_v, nb_v):
    cid = jax.lax.axis_index("c")
    sid = jax.lax.axis_index("s")
    wid = sid * 2 + cid
    b = wid // 4
    q = wid % 4
    pltpu.sync_copy(idxf_hbm.at[b], idx_v)
    lanes = jax.lax.iota(jnp.int32, 16)
    for cc in range(16):
        ch = q * 16 + cc
        pltpu.sync_copy(x_hbm.at[b, ch], table_v)

        def blk_body(blk, carry, ch=ch):
            def chunk_body(ii, carry2, blk=blk):
                n = blk * 128 + ii
                off = ii * 16
                idxv = idx_v[pl.ds(n * 16, 16)]
                g = plsc.load_gather(table_v, [idxv])
                cen = plsc.load_gather(table_v, [lanes * 0 + n])
                cen_v[pl.ds(off, 16)] = cen
                nb_v[pl.ds(off, 16)] = g - cen
                return carry2

            jax.lax.fori_loop(0, 128, chunk_body, 0)
            pltpu.sync_copy(
                cen_v, out_hbm.at[b, ch, pl.ds(blk * 2048, 2048)])
            pltpu.sync_copy(
                nb_v, out_hbm.at[b, _D + ch, pl.ds(blk * 2048, 2048)])
            return carry

        jax.lax.fori_loop(0, 16, blk_body, 0)


def _build_topk_call(B):
    return pl.pallas_call(
        _topk_kernel,
        grid=(B, _N // _R),
        in_specs=[
            pl.BlockSpec((1, _N, _D), lambda b, r: (b, 0, 0)),
            pl.BlockSpec((1, _D, _R), lambda b, r: (b, 0, r)),
            pl.BlockSpec((1, _N, 1), lambda b, r: (b, 0, 0)),
            pl.BlockSpec((1, 1, _R), lambda b, r: (b, 0, r)),
        ],
        out_specs=[
            pl.BlockSpec((1, _K, _R), lambda b, r: (b, 0, r)),
        ],
        out_shape=[
            jax.ShapeDtypeStruct((B, _K, _N), jnp.int32),
        ],
    )


def _build_sc_call(B):
    mesh = plsc.VectorSubcoreMesh(core_axis_name="c", subcore_axis_name="s")
    return functools.partial(
        pl.kernel,
        out_type=jax.ShapeDtypeStruct((B, 2 * _D, _NK), jnp.float32),
        mesh=mesh,
        scratch_types=[
            pltpu.VMEM((2048,), jnp.float32),
            pltpu.VMEM((_NK,), jnp.int32),
            pltpu.VMEM((2048,), jnp.float32),
            pltpu.VMEM((2048,), jnp.float32),
        ],
        compiler_params=pltpu.CompilerParams(needs_layout_passes=False),
    )(_sc_gather_kernel)


def kernel(point_cloud):
    B, D, N = point_cloud.shape
    xt = jnp.transpose(point_cloud, (0, 2, 1))
    sq = jnp.sum(xt * xt, axis=-1)
    sqc = sq[:, :, None]
    sqr = sq[:, None, :]
    (idx,) = _build_topk_call(B)(xt, point_cloud, sqc, sqr)
    idx_flat = jnp.transpose(idx, (0, 2, 1)).reshape(B, _NK)
    edge_flat = _build_sc_call(B)(point_cloud, idx_flat)
    edge_feature = edge_flat.reshape(B, 2 * D, N, _K)
    return (edge_feature, idx_flat)


# speedup vs baseline: 2.0086x; 2.0086x over previous
import jax
import jax.numpy as jnp
from jax.experimental import pallas as pl
from jax.experimental.pallas import tpu as pltpu

_K = 16
_KP1 = 17
_N = 2048
_D = 64
_R = 512


def _edge_kernel(xt_ref, xq_ref, xs_ref, sqc_ref, sqr_ref, edge_ref, idx_ref):
    xt = xt_ref[0]
    xq = xq_ref[0]
    mm = jax.lax.dot_general(
        xt, xq, (((1,), (0,)), ((), ())),
        preferred_element_type=jnp.float32)
    d2 = (sqc_ref[0] + sqr_ref[0]) - 2.0 * mm
    dist = jnp.sqrt(jnp.maximum(d2, 0.0))
    iota = jax.lax.broadcasted_iota(jnp.int32, (_N, _R), 0)
    iota8 = jax.lax.broadcasted_iota(jnp.int32, (8, _R), 0)
    xs = xs_ref[0]
    edge_ref[0, 0:_D, :, :] = jnp.broadcast_to(xq[:, None, :], (_D, _K, _R))
    dT = dist
    nch = _N // 8
    nchain = 4
    cpc = nch // nchain
    for t in range(_KP1):
        vs, cs = [], []
        for ch in range(nchain):
            b0 = ch * cpc
            vacc = dT[b0 * 8:(b0 + 1) * 8, :]
            cacc = jnp.full((8, _R), b0, jnp.int32)
            for c in range(b0 + 1, b0 + cpc):
                v = dT[c * 8:(c + 1) * 8, :]
                take = v < vacc
                vacc = jnp.where(take, v, vacc)
                cacc = jnp.where(take, c, cacc)
            vs.append(vacc)
            cs.append(cacc)
        while len(vs) > 1:
            take = vs[1] < vs[0]
            vs = [jnp.where(take, vs[1], vs[0])] + vs[2:]
            cs = [jnp.where(take, cs[1], cs[0])] + cs[2:]
        fidx = cs[0] * 8 + iota8
        vcur, icur = vs[0], fidx
        for sh in (4, 2, 1):
            vb = pltpu.roll(vcur, 8 - sh, axis=0)
            ib = pltpu.bitcast(
                pltpu.roll(pltpu.bitcast(icur, jnp.float32), 8 - sh, axis=0),
                jnp.int32)
            take = (vb < vcur) | ((vb == vcur) & (ib < icur))
            vcur = jnp.where(take, vb, vcur)
            icur = jnp.where(take, ib, icur)
        j = icur[0:1, :]
        hit = iota == j
        if t >= 1:
            oh = hit.astype(jnp.bfloat16)
            nb2 = jax.lax.dot_general(
                xs, oh, (((1,), (0,)), ((), ())),
                preferred_element_type=jnp.float32)
            nb = nb2[0:_D] + nb2[_D:2 * _D]
            edge_ref[0, _D:2 * _D, t - 1, :] = nb - xq
            idx_ref[0, t - 1:t, :] = j
        if t < _KP1 - 1:
            dT = jnp.where(hit, jnp.float32(jnp.inf), dT)


def _build_call(B):
    return pl.pallas_call(
        _edge_kernel,
        grid=(B, _N // _R),
        in_specs=[
            pl.BlockSpec((1, _N, _D), lambda b, r: (b, 0, 0)),
            pl.BlockSpec((1, _D, _R), lambda b, r: (b, 0, r)),
            pl.BlockSpec((1, 2 * _D, _N), lambda b, r: (b, 0, 0)),
            pl.BlockSpec((1, _N, 1), lambda b, r: (b, 0, 0)),
            pl.BlockSpec((1, 1, _R), lambda b, r: (b, 0, r)),
        ],
        out_specs=[
            pl.BlockSpec((1, 2 * _D, _K, _R), lambda b, r: (b, 0, 0, r)),
            pl.BlockSpec((1, _K, _R), lambda b, r: (b, 0, r)),
        ],
        out_shape=[
            jax.ShapeDtypeStruct((B, 2 * _D, _K, _N), jnp.float32),
            jax.ShapeDtypeStruct((B, _K, _N), jnp.int32),
        ],
    )


def kernel(point_cloud):
    B, D, N = point_cloud.shape
    xt = jnp.transpose(point_cloud, (0, 2, 1))
    sq = jnp.sum(xt * xt, axis=-1)
    hi = point_cloud.astype(jnp.bfloat16)
    lo = (point_cloud - hi.astype(jnp.float32)).astype(jnp.bfloat16)
    xs = jnp.concatenate([hi, lo], axis=1)
    sqc = sq[:, :, None]
    sqr = sq[:, None, :]
    edge, idx = _build_call(B)(xt, point_cloud, xs, sqc, sqr)
    edge_feature = jnp.transpose(edge, (0, 1, 3, 2))
    idx_out = jnp.transpose(idx, (0, 2, 1)).reshape(B, N * _K)
    return (edge_feature, idx_out)
